# baseline (speedup 1.0000x reference)
import jax
import jax.numpy as jnp
from jax import lax
from jax.experimental import pallas as pl
from jax.experimental.pallas import tpu as pltpu

_H = 28
_W = 28
_HW = _H * _W
_C00 = (((0,), (0,)), ((), ()))


def _dot(a, b):
    return lax.dot_general(a, b, _C00, preferred_element_type=jnp.float32)


def _inception_kernel(x_ref, w1_ref, b1cm_ref, w23_ref, b23cm_ref, wblk_ref,
                      bcvcm_ref, w4_ref, b4cm_ref, cmul_ref, cadd_ref,
                      fmul_ref, o_ref):
    c1 = w1_ref.shape[1]
    cc = wblk_ref.shape[-1]
    wblk = wblk_ref[...]

    for img in range(x_ref.shape[0]):
        xb = x_ref[img].astype(jnp.bfloat16)

        o_ref[img, 0:c1, :] = jnp.maximum(
            _dot(w1_ref[...], xb) + b1cm_ref[...], 0.0)

        y23 = jnp.maximum(_dot(w23_ref[...], xb) + b23cm_ref[...], 0.0)
        y23 = y23.astype(jnp.bfloat16)

        tl = pltpu.roll(y23, 1, axis=1) * cmul_ref[0:1, :]
        tr = pltpu.roll(y23, _HW - 1, axis=1) * cmul_ref[1:2, :]
        z = []
        for dy in range(3):
            a = _dot(wblk[dy, 0], tl)
            a = a + _dot(wblk[dy, 1], y23)
            a = a + _dot(wblk[dy, 2], tr)
            z.append(a)
        acc = bcvcm_ref[...] + z[1]
        acc = acc + pltpu.roll(z[0], _W, axis=1) * fmul_ref[0:1, :]
        acc = acc + pltpu.roll(z[2], _HW - _W, axis=1) * fmul_ref[1:2, :]
        o_ref[img, c1:c1 + cc, :] = jnp.maximum(acc, 0.0)

        h = jnp.maximum(pltpu.roll(xb, 1, axis=1) + cadd_ref[0:1, :],
                        pltpu.roll(xb, _HW - 1, axis=1) + cadd_ref[1:2, :])
        h = jnp.maximum(h, xb)
        p2 = jnp.maximum(pltpu.roll(h, _W, axis=1) + cadd_ref[2:3, :],
                         pltpu.roll(h, _HW - _W, axis=1) + cadd_ref[3:4, :])
        p2 = jnp.maximum(p2, h)
        o_ref[img, c1 + cc:, :] = jnp.maximum(
            _dot(w4_ref[...], p2) + b4cm_ref[...], 0.0)


def kernel(x_nchw, b1_w, b2_red_w, b3_red_w, fused1x1_w, fused1x1_bias,
           b2_conv_w, b2_conv_bias, b3_conv_w, b3_conv_bias,
           b4_proj_w, b4_proj_bias):
    n, cin, h, w = x_nchw.shape
    hw = h * w
    c1 = b1_w.shape[1]
    c2r = b2_red_w.shape[1]
    c3r = b3_red_w.shape[1]
    c2 = b2_conv_w.shape[-1]
    c3 = b3_conv_w.shape[-1]
    c4 = b4_proj_w.shape[-1]
    cr = c2r + c3r
    cc = c2 + c3
    cout = c1 + cc + c4

    x = x_nchw.reshape(n, cin, hw)

    w23 = jnp.concatenate([b2_red_w, b3_red_w], axis=1)
    wblk = jnp.zeros((3, 3, cr, cc), jnp.bfloat16)
    wblk = wblk.at[:, :, :c2r, :c2].set(b2_conv_w)
    wblk = wblk.at[:, :, c2r:, c2:].set(b3_conv_w)
    b1cm = jnp.broadcast_to(fused1x1_bias[:c1, None], (c1, hw))
    b23cm = jnp.broadcast_to(fused1x1_bias[c1:, None], (cr, hw))
    bcvcm = jnp.broadcast_to(
        jnp.concatenate([b2_conv_bias, b3_conv_bias])[:, None], (cc, hw))
    b4cm = jnp.broadcast_to(b4_proj_bias[:, None], (c4, hw))

    p = jnp.arange(hw)
    col = p % w
    lvalid = col != 0
    rvalid = col != (w - 1)
    tvalid = p >= w
    bvalid = p < (hw - w)
    cmul = jnp.stack([lvalid, rvalid]).astype(jnp.bfloat16)
    ninf = jnp.float32(-jnp.inf)
    cadd = jnp.stack([jnp.where(lvalid, 0.0, ninf),
                      jnp.where(rvalid, 0.0, ninf),
                      jnp.where(tvalid, 0.0, ninf),
                      jnp.where(bvalid, 0.0, ninf)]).astype(jnp.bfloat16)
    fmul = jnp.stack([tvalid, bvalid]).astype(jnp.float32)

    ipg = 4 if n % 4 == 0 else 1
    out = pl.pallas_call(
        _inception_kernel,
        out_shape=jax.ShapeDtypeStruct((n, cout, hw), jnp.float32),
        grid=(n // ipg,),
        in_specs=[
            pl.BlockSpec((ipg, cin, hw), lambda i: (i, 0, 0)),
            pl.BlockSpec((cin, c1), lambda i: (0, 0)),
            pl.BlockSpec((c1, hw), lambda i: (0, 0)),
            pl.BlockSpec((cin, cr), lambda i: (0, 0)),
            pl.BlockSpec((cr, hw), lambda i: (0, 0)),
            pl.BlockSpec((3, 3, cr, cc), lambda i: (0, 0, 0, 0)),
            pl.BlockSpec((cc, hw), lambda i: (0, 0)),
            pl.BlockSpec((cin, c4), lambda i: (0, 0)),
            pl.BlockSpec((c4, hw), lambda i: (0, 0)),
            pl.BlockSpec((2, hw), lambda i: (0, 0)),
            pl.BlockSpec((4, hw), lambda i: (0, 0)),
            pl.BlockSpec((2, hw), lambda i: (0, 0)),
        ],
        out_specs=pl.BlockSpec((ipg, cout, hw), lambda i: (i, 0, 0)),
        compiler_params=pltpu.CompilerParams(
            dimension_semantics=("parallel",),
            vmem_limit_bytes=64 * 1024 * 1024,
        ),
    )(x, b1_w, b1cm, w23, b23cm, wblk, bcvcm, b4_proj_w, b4cm,
      cmul, cadd, fmul)
    return out.reshape(n, cout, h, w)


# speedup vs baseline: 1.1857x; 1.1857x over previous
import jax
import jax.numpy as jnp
from jax import lax
from jax.experimental import pallas as pl
from jax.experimental.pallas import tpu as pltpu

_H = 28
_W = 28
_HW = _H * _W
_C10 = (((1,), (0,)), ((), ()))
_TAPS = [(oy, ox) for oy in (-1, 0, 1) for ox in (-1, 0, 1) if (oy, ox) != (0, 0)]


def _dot(a, b):
    return lax.dot_general(a, b, _C10, preferred_element_type=jnp.float32)


def _inception_kernel(x_ref, w123_ref, b123cm_ref, wblk_ref, bcvcm_ref,
                      w4_ref, b4cm_ref, mt_ref, cadd_ref, o_ref):
    c1 = w123_ref.shape[0] - wblk_ref.shape[2]
    cc = wblk_ref.shape[1]

    xb = x_ref[0].astype(jnp.bfloat16)

    y123 = jnp.maximum(_dot(w123_ref[...], xb) + b123cm_ref[...], 0.0)
    o_ref[0, 0:c1, :] = y123[:c1]
    y23 = y123[c1:].astype(jnp.bfloat16)

    acc = bcvcm_ref[...] + _dot(wblk_ref[4], y23)
    for t, (oy, ox) in enumerate(_TAPS):
        d = oy * _W + ox
        tap = pltpu.roll(y23, (-d) % _HW, axis=1) * mt_ref[t:t + 1, :]
        wi = t if t < 4 else t + 1
        acc = acc + _dot(wblk_ref[wi], tap)
    o_ref[0, c1:c1 + cc, :] = jnp.maximum(acc, 0.0)

    h = jnp.maximum(pltpu.roll(xb, 1, axis=1) + cadd_ref[0:1, :],
                    pltpu.roll(xb, _HW - 1, axis=1) + cadd_ref[1:2, :])
    h = jnp.maximum(h, xb)
    p2 = jnp.maximum(pltpu.roll(h, _W, axis=1) + cadd_ref[2:3, :],
                     pltpu.roll(h, _HW - _W, axis=1) + cadd_ref[3:4, :])
    p2 = jnp.maximum(p2, h)
    o_ref[0, c1 + cc:, :] = jnp.maximum(_dot(w4_ref[...], p2)
                                        + b4cm_ref[...], 0.0)


def kernel(x_nchw, b1_w, b2_red_w, b3_red_w, fused1x1_w, fused1x1_bias,
           b2_conv_w, b2_conv_bias, b3_conv_w, b3_conv_bias,
           b4_proj_w, b4_proj_bias):
    n, cin, h, w = x_nchw.shape
    hw = h * w
    c1 = b1_w.shape[1]
    c2r = b2_red_w.shape[1]
    c3r = b3_red_w.shape[1]
    c2 = b2_conv_w.shape[-1]
    c3 = b3_conv_w.shape[-1]
    c4 = b4_proj_w.shape[-1]
    cr = c2r + c3r
    cc = c2 + c3
    cout = c1 + cc + c4

    x = x_nchw.reshape(n, cin, hw)

    w123 = fused1x1_w.T
    wblk = jnp.zeros((3, 3, cr, cc), jnp.bfloat16)
    wblk = wblk.at[:, :, :c2r, :c2].set(b2_conv_w)
    wblk = wblk.at[:, :, c2r:, c2:].set(b3_conv_w)
    wblkt = jnp.transpose(wblk, (0, 1, 3, 2)).reshape(9, cc, cr)
    w4t = b4_proj_w.T
    b123cm = jnp.broadcast_to(fused1x1_bias[:, None], (c1 + cr, hw))
    bcvcm = jnp.broadcast_to(
        jnp.concatenate([b2_conv_bias, b3_conv_bias])[:, None], (cc, hw))
    b4cm = jnp.broadcast_to(b4_proj_bias[:, None], (c4, hw))

    p = jnp.arange(hw)
    col = p % w
    row = p // w
    lvalid = col != 0
    rvalid = col != (w - 1)
    tvalid = p >= w
    bvalid = p < (hw - w)
    taps = [(oy, ox) for oy in (-1, 0, 1) for ox in (-1, 0, 1)
            if (oy, ox) != (0, 0)]
    mt = jnp.stack(
        [((row + oy >= 0) & (row + oy < h) & (col + ox >= 0) & (col + ox < w))
         for (oy, ox) in taps]).astype(jnp.bfloat16)
    ninf = jnp.float32(-jnp.inf)
    cadd = jnp.stack([jnp.where(lvalid, 0.0, ninf),
                      jnp.where(rvalid, 0.0, ninf),
                      jnp.where(tvalid, 0.0, ninf),
                      jnp.where(bvalid, 0.0, ninf)]).astype(jnp.bfloat16)

    out = pl.pallas_call(
        _inception_kernel,
        out_shape=jax.ShapeDtypeStruct((n, cout, hw), jnp.float32),
        grid=(n,),
        in_specs=[
            pl.BlockSpec((1, cin, hw), lambda i: (i, 0, 0)),
            pl.BlockSpec((c1 + cr, cin), lambda i: (0, 0)),
            pl.BlockSpec((c1 + cr, hw), lambda i: (0, 0)),
            pl.BlockSpec((9, cc, cr), lambda i: (0, 0, 0)),
            pl.BlockSpec((cc, hw), lambda i: (0, 0)),
            pl.BlockSpec((c4, cin), lambda i: (0, 0)),
            pl.BlockSpec((c4, hw), lambda i: (0, 0)),
            pl.BlockSpec((8, hw), lambda i: (0, 0)),
            pl.BlockSpec((4, hw), lambda i: (0, 0)),
        ],
        out_specs=pl.BlockSpec((1, cout, hw), lambda i: (i, 0, 0)),
        compiler_params=pltpu.CompilerParams(
            dimension_semantics=("parallel",),
            vmem_limit_bytes=64 * 1024 * 1024,
        ),
    )(x, w123, b123cm, wblkt, bcvcm, w4t, b4cm, mt, cadd)
    return out.reshape(n, cout, h, w)


# speedup vs baseline: 1.2126x; 1.0226x over previous
import jax
import jax.numpy as jnp
import numpy as np
from jax import lax
from jax.experimental import pallas as pl
from jax.experimental.pallas import tpu as pltpu

_H = 28
_W = 28
_HW = _H * _W
_C10 = (((1,), (0,)), ((), ()))
_TAPS = [(oy, ox) for oy in (-1, 0, 1) for ox in (-1, 0, 1) if (oy, ox) != (0, 0)]


def _np_masks():
    p = np.arange(_HW)
    col = p % _W
    row = p // _W
    mt = np.stack([((row + oy >= 0) & (row + oy < _H)
                    & (col + ox >= 0) & (col + ox < _W))
                   for (oy, ox) in _TAPS]).astype(np.float32)
    ninf = np.float32(-np.inf)
    cadd = np.stack([np.where(col != 0, 0.0, ninf),
                     np.where(col != _W - 1, 0.0, ninf),
                     np.where(p >= _W, 0.0, ninf),
                     np.where(p < _HW - _W, 0.0, ninf)]).astype(np.float32)
    return (np.asarray(mt, dtype=jnp.bfloat16),
            np.asarray(cadd, dtype=jnp.bfloat16))


def _dot(a, b):
    return lax.dot_general(a, b, _C10, preferred_element_type=jnp.float32)


def _prep_kernel(fw_ref, w2_ref, w3_ref, w4_ref, fb_ref, b2b_ref, b3b_ref,
                 b4b_ref, w123t_ref, wblkt_ref, w4t_ref, b123cm_ref,
                 bcvcm_ref, b4cm_ref):
    c2r = w2_ref.shape[1]
    c2 = w2_ref.shape[2]
    w123t_ref[...] = jnp.transpose(fw_ref[...])
    w4t_ref[...] = jnp.transpose(w4_ref[...])
    wblkt_ref[...] = jnp.zeros_like(wblkt_ref)
    for i in range(9):
        wblkt_ref[i, 0:c2, 0:c2r] = jnp.transpose(w2_ref[i])
        wblkt_ref[i, c2:, c2r:] = jnp.transpose(w3_ref[i])
    hw = b123cm_ref.shape[-1]
    b123cm_ref[...] = jnp.broadcast_to(jnp.transpose(fb_ref[...]),
                                       b123cm_ref.shape)
    bcvcm_ref[0:c2, :] = jnp.broadcast_to(jnp.transpose(b2b_ref[...]),
                                          (c2, hw))
    bcvcm_ref[c2:, :] = jnp.broadcast_to(jnp.transpose(b3b_ref[...]),
                                         (bcvcm_ref.shape[0] - c2, hw))
    b4cm_ref[...] = jnp.broadcast_to(jnp.transpose(b4b_ref[...]),
                                     b4cm_ref.shape)


def _inception_kernel(x_ref, w123_ref, b123cm_ref, wblk_ref, bcvcm_ref,
                      w4_ref, b4cm_ref, mt_ref, cadd_ref, o_ref):
    c1 = w123_ref.shape[0] - wblk_ref.shape[2]
    cc = wblk_ref.shape[1]

    xb = x_ref[0].astype(jnp.bfloat16)

    y123 = jnp.maximum(_dot(w123_ref[...], xb) + b123cm_ref[...], 0.0)
    o_ref[0, 0:c1, :] = y123[:c1]
    y23 = y123[c1:].astype(jnp.bfloat16)

    acc = bcvcm_ref[...] + _dot(wblk_ref[4], y23)
    for t, (oy, ox) in enumerate(_TAPS):
        d = oy * _W + ox
        tap = pltpu.roll(y23, (-d) % _HW, axis=1) * mt_ref[t:t + 1, :]
        wi = t if t < 4 else t + 1
        acc = acc + _dot(wblk_ref[wi], tap)
    o_ref[0, c1:c1 + cc, :] = jnp.maximum(acc, 0.0)

    h = jnp.maximum(pltpu.roll(xb, 1, axis=1) + cadd_ref[0:1, :],
                    pltpu.roll(xb, _HW - 1, axis=1) + cadd_ref[1:2, :])
    h = jnp.maximum(h, xb)
    p2 = jnp.maximum(pltpu.roll(h, _W, axis=1) + cadd_ref[2:3, :],
                     pltpu.roll(h, _HW - _W, axis=1) + cadd_ref[3:4, :])
    p2 = jnp.maximum(p2, h)
    o_ref[0, c1 + cc:, :] = jnp.maximum(_dot(w4_ref[...], p2)
                                        + b4cm_ref[...], 0.0)


def kernel(x_nchw, b1_w, b2_red_w, b3_red_w, fused1x1_w, fused1x1_bias,
           b2_conv_w, b2_conv_bias, b3_conv_w, b3_conv_bias,
           b4_proj_w, b4_proj_bias):
    n, cin, h, w = x_nchw.shape
    hw = h * w
    c1 = b1_w.shape[1]
    c2r = b2_red_w.shape[1]
    c2 = b2_conv_w.shape[-1]
    c3 = b3_conv_w.shape[-1]
    c4 = b4_proj_w.shape[-1]
    cr = fused1x1_w.shape[1] - c1
    cc = c2 + c3
    cout = c1 + cc + c4

    x = x_nchw.reshape(n, cin, hw)

    w123t, wblkt, w4t, b123cm, bcvcm, b4cm = pl.pallas_call(
        _prep_kernel,
        out_shape=(
            jax.ShapeDtypeStruct((c1 + cr, cin), jnp.bfloat16),
            jax.ShapeDtypeStruct((9, cc, cr), jnp.bfloat16),
            jax.ShapeDtypeStruct((c4, cin), jnp.bfloat16),
            jax.ShapeDtypeStruct((c1 + cr, hw), jnp.float32),
            jax.ShapeDtypeStruct((cc, hw), jnp.float32),
            jax.ShapeDtypeStruct((c4, hw), jnp.float32),
        ),
        compiler_params=pltpu.CompilerParams(
            vmem_limit_bytes=64 * 1024 * 1024),
    )(fused1x1_w, b2_conv_w.reshape(9, c2r, c2),
      b3_conv_w.reshape(9, cr - c2r, c3), b4_proj_w,
      fused1x1_bias.reshape(1, c1 + cr), b2_conv_bias.reshape(1, c2),
      b3_conv_bias.reshape(1, c3), b4_proj_bias.reshape(1, c4))

    mt, cadd = _np_masks()

    out = pl.pallas_call(
        _inception_kernel,
        out_shape=jax.ShapeDtypeStruct((n, cout, hw), jnp.float32),
        grid=(n,),
        in_specs=[
            pl.BlockSpec((1, cin, hw), lambda i: (i, 0, 0)),
            pl.BlockSpec((c1 + cr, cin), lambda i: (0, 0)),
            pl.BlockSpec((c1 + cr, hw), lambda i: (0, 0)),
            pl.BlockSpec((9, cc, cr), lambda i: (0, 0, 0)),
            pl.BlockSpec((cc, hw), lambda i: (0, 0)),
            pl.BlockSpec((c4, cin), lambda i: (0, 0)),
            pl.BlockSpec((c4, hw), lambda i: (0, 0)),
            pl.BlockSpec((8, hw), lambda i: (0, 0)),
            pl.BlockSpec((4, hw), lambda i: (0, 0)),
        ],
        out_specs=pl.BlockSpec((1, cout, hw), lambda i: (i, 0, 0)),
        compiler_params=pltpu.CompilerParams(
            dimension_semantics=("parallel",),
            vmem_limit_bytes=64 * 1024 * 1024,
        ),
    )(x, w123t, b123cm, wblkt, bcvcm, w4t, b4cm, mt, cadd)
    return out.reshape(n, cout, h, w)


# speedup vs baseline: 2.2564x; 1.8609x over previous
import jax
import jax.numpy as jnp
from jax import lax
from jax.experimental import pallas as pl
from jax.experimental.pallas import tpu as pltpu

_C10 = (((1,), (0,)), ((), ()))


def _dot(a, b):
    return lax.dot_general(a, b, _C10, preferred_element_type=jnp.float32)


def _prep_kernel(fw_ref, w2_ref, w3_ref, w4_ref, fb_ref, b2b_ref, b3b_ref,
                 b4b_ref, w23_ref, b23_ref, w1e_ref, wcv_ref, w4e_ref,
                 bf_ref):
    c1 = fw_ref.shape[1] - w23_ref.shape[1]
    c2r, c2 = w2_ref.shape[1], w2_ref.shape[2]
    c3 = w3_ref.shape[2]
    w23_ref[...] = fw_ref[:, c1:]
    b23_ref[...] = fb_ref[:, c1:]
    w1e_ref[...] = jnp.zeros_like(w1e_ref)
    w1e_ref[:, 0:c1] = fw_ref[:, 0:c1]
    wcv_ref[...] = jnp.zeros_like(wcv_ref)
    for i in range(9):
        wcv_ref[i, 0:c2r, c1:c1 + c2] = w2_ref[i]
        wcv_ref[i, c2r:, c1 + c2:c1 + c2 + c3] = w3_ref[i]
    w4e_ref[...] = jnp.zeros_like(w4e_ref)
    w4e_ref[:, c1 + c2 + c3:] = w4_ref[...]
    bf_ref[0:1, 0:c1] = fb_ref[:, 0:c1]
    bf_ref[0:1, c1:c1 + c2] = b2b_ref[...]
    bf_ref[0:1, c1 + c2:c1 + c2 + c3] = b3b_ref[...]
    bf_ref[0:1, c1 + c2 + c3:] = b4b_ref[...]


def _shift(v, ox, sh, fill):
    if ox == 0:
        return v
    z = jnp.full((sh, v.shape[1]), fill, v.dtype)
    if ox > 0:
        return jnp.concatenate([v[sh:], z], axis=0)
    return jnp.concatenate([z, v[:-sh]], axis=0)


def _inception_kernel(xm_ref, xc_ref, xp_ref, w23_ref, b23_ref, w1e_ref,
                      wcv_ref, w4e_ref, bf_ref, o_ref):
    i = pl.program_id(0)
    nh = pl.num_programs(0)
    sh = xc_ref.shape[1] // 28
    ninf = -jnp.inf

    xbm = xm_ref[0].astype(jnp.bfloat16)
    xbc = xc_ref[0].astype(jnp.bfloat16)
    xbp = xp_ref[0].astype(jnp.bfloat16)

    w23 = w23_ref[...]
    b23 = b23_ref[...]

    def red(xb):
        return jnp.maximum(_dot(xb, w23) + b23, 0.0).astype(jnp.bfloat16)

    yc = red(xbc)
    ym = red(xbm)
    yp = red(xbp)
    ym = jnp.where(i == 0, jnp.zeros_like(ym), ym)
    yp = jnp.where(i == nh - 1, jnp.zeros_like(yp), yp)

    acc = _dot(xbc, w1e_ref[...]) + bf_ref[...]
    t = 0
    for b in (ym, yc, yp):
        for ox in (-1, 0, 1):
            acc = acc + _dot(_shift(b, ox, sh, 0), wcv_ref[t])
            t += 1

    def hmax(v):
        return jnp.maximum(v, jnp.maximum(_shift(v, 1, sh, ninf),
                                          _shift(v, -1, sh, ninf)))

    hm = jnp.where(i == 0, jnp.full_like(xbm, ninf), hmax(xbm))
    hp = jnp.where(i == nh - 1, jnp.full_like(xbp, ninf), hmax(xbp))
    pooled = jnp.maximum(hmax(xbc), jnp.maximum(hm, hp))
    acc = acc + _dot(pooled, w4e_ref[...])

    o_ref[0] = jnp.maximum(acc, 0.0)


def kernel(x_nchw, b1_w, b2_red_w, b3_red_w, fused1x1_w, fused1x1_bias,
           b2_conv_w, b2_conv_bias, b3_conv_w, b3_conv_bias,
           b4_proj_w, b4_proj_bias):
    n, cin, h, w = x_nchw.shape
    c1 = b1_w.shape[1]
    c2r = b2_red_w.shape[1]
    c2 = b2_conv_w.shape[-1]
    c3 = b3_conv_w.shape[-1]
    c4 = b4_proj_w.shape[-1]
    cr = fused1x1_w.shape[1] - c1
    cout = c1 + c2 + c3 + c4
    wn = w * n

    x3 = x_nchw.transpose(2, 3, 0, 1).reshape(h, wn, cin)

    w23, b23, w1e, wcv, w4e, bf = pl.pallas_call(
        _prep_kernel,
        out_shape=(
            jax.ShapeDtypeStruct((cin, cr), jnp.bfloat16),
            jax.ShapeDtypeStruct((1, cr), jnp.float32),
            jax.ShapeDtypeStruct((cin, cout), jnp.bfloat16),
            jax.ShapeDtypeStruct((9, cr, cout), jnp.bfloat16),
            jax.ShapeDtypeStruct((cin, cout), jnp.bfloat16),
            jax.ShapeDtypeStruct((1, cout), jnp.float32),
        ),
        compiler_params=pltpu.CompilerParams(
            vmem_limit_bytes=64 * 1024 * 1024),
    )(fused1x1_w, b2_conv_w.reshape(9, c2r, c2),
      b3_conv_w.reshape(9, cr - c2r, c3), b4_proj_w,
      fused1x1_bias.reshape(1, c1 + cr), b2_conv_bias.reshape(1, c2),
      b3_conv_bias.reshape(1, c3), b4_proj_bias.reshape(1, c4))

    out3 = pl.pallas_call(
        _inception_kernel,
        out_shape=jax.ShapeDtypeStruct((h, wn, cout), jnp.float32),
        grid=(h,),
        in_specs=[
            pl.BlockSpec((1, wn, cin), lambda i: (jnp.maximum(i - 1, 0), 0, 0)),
            pl.BlockSpec((1, wn, cin), lambda i: (i, 0, 0)),
            pl.BlockSpec((1, wn, cin),
                         lambda i, _h=h: (jnp.minimum(i + 1, _h - 1), 0, 0)),
            pl.BlockSpec((cin, cr), lambda i: (0, 0)),
            pl.BlockSpec((1, cr), lambda i: (0, 0)),
            pl.BlockSpec((cin, cout), lambda i: (0, 0)),
            pl.BlockSpec((9, cr, cout), lambda i: (0, 0, 0)),
            pl.BlockSpec((cin, cout), lambda i: (0, 0)),
            pl.BlockSpec((1, cout), lambda i: (0, 0)),
        ],
        out_specs=pl.BlockSpec((1, wn, cout), lambda i: (i, 0, 0)),
        compiler_params=pltpu.CompilerParams(
            dimension_semantics=("parallel",),
            vmem_limit_bytes=64 * 1024 * 1024,
        ),
    )(x3, x3, x3, w23, b23, w1e, wcv, w4e, bf)

    return out3.reshape(h, w, n, cout).transpose(2, 3, 0, 1)


# speedup vs baseline: 2.5362x; 1.1240x over previous
import jax
import jax.numpy as jnp
from jax import lax
from jax.experimental import pallas as pl
from jax.experimental.pallas import tpu as pltpu

_C10 = (((1,), (0,)), ((), ()))


def _dot(a, b):
    return lax.dot_general(a, b, _C10, preferred_element_type=jnp.float32)


def _prep_kernel(fw_ref, w2_ref, w3_ref, w4_ref, fb_ref, b2b_ref, b3b_ref,
                 b4b_ref, w23_ref, b23_ref, w1e_ref, wcv_ref, w4e_ref,
                 bf_ref):
    c1 = fw_ref.shape[1] - w23_ref.shape[1]
    c2r, c2 = w2_ref.shape[1], w2_ref.shape[2]
    c3 = w3_ref.shape[2]
    w23_ref[...] = fw_ref[:, c1:]
    b23_ref[...] = fb_ref[:, c1:]
    w1e_ref[...] = jnp.zeros_like(w1e_ref)
    w1e_ref[:, 0:c1] = fw_ref[:, 0:c1]
    wcv_ref[...] = jnp.zeros_like(wcv_ref)
    for i in range(9):
        wcv_ref[i, 0:c2r, c1:c1 + c2] = w2_ref[i]
        wcv_ref[i, c2r:, c1 + c2:c1 + c2 + c3] = w3_ref[i]
    w4e_ref[...] = jnp.zeros_like(w4e_ref)
    w4e_ref[:, c1 + c2 + c3:] = w4_ref[...]
    bf_ref[0:1, 0:c1] = fb_ref[:, 0:c1]
    bf_ref[0:1, c1:c1 + c2] = b2b_ref[...]
    bf_ref[0:1, c1 + c2:c1 + c2 + c3] = b3b_ref[...]
    bf_ref[0:1, c1 + c2 + c3:] = b4b_ref[...]


def _shift(v, ox, sh, fill):
    if ox == 0:
        return v
    z = jnp.full((sh, v.shape[1]), fill, v.dtype)
    if ox > 0:
        return jnp.concatenate([v[sh:], z], axis=0)
    return jnp.concatenate([z, v[:-sh]], axis=0)


def _inception_kernel(x0_ref, xp_ref, w23_ref, b23_ref, w1e_ref, wcv_ref,
                      w4e_ref, bf_ref, o_ref, xring, yring):
    i = pl.program_id(0)
    nh = pl.num_programs(0)
    sh = o_ref.shape[1] // 28
    ninf = -jnp.inf
    w23 = w23_ref[...]
    b23 = b23_ref[...]

    def red(xb):
        return jnp.maximum(_dot(xb, w23) + b23, 0.0).astype(jnp.bfloat16)

    @pl.when(i == 0)
    def _():
        xb0 = x0_ref[0].astype(jnp.bfloat16)
        xring[0] = xb0
        yring[0] = red(xb0)

    s_next = lax.rem(i + 1, 3)
    xb1 = xp_ref[0].astype(jnp.bfloat16)
    xring[s_next] = xb1
    yring[s_next] = red(xb1)

    s_prev = lax.rem(i + 2, 3)
    s_cur = lax.rem(i, 3)
    first = i == 0
    last = i == nh - 1

    yc = yring[s_cur]
    ym = jnp.where(first, jnp.zeros_like(yc), yring[s_prev])
    yp = jnp.where(last, jnp.zeros_like(yc), yring[s_next])
    xbc = xring[s_cur]

    acc = _dot(xbc, w1e_ref[...]) + bf_ref[...]
    t = 0
    for b in (ym, yc, yp):
        for ox in (-1, 0, 1):
            acc = acc + _dot(_shift(b, ox, sh, 0), wcv_ref[t])
            t += 1

    def hmax(v):
        return jnp.maximum(v, jnp.maximum(_shift(v, 1, sh, ninf),
                                          _shift(v, -1, sh, ninf)))

    hm = jnp.where(first, jnp.full_like(xbc, ninf), hmax(xring[s_prev]))
    hp = jnp.where(last, jnp.full_like(xbc, ninf), hmax(xb1))
    pooled = jnp.maximum(hmax(xbc), jnp.maximum(hm, hp))
    acc = acc + _dot(pooled, w4e_ref[...])

    o_ref[0] = jnp.maximum(acc, 0.0)


def kernel(x_nchw, b1_w, b2_red_w, b3_red_w, fused1x1_w, fused1x1_bias,
           b2_conv_w, b2_conv_bias, b3_conv_w, b3_conv_bias,
           b4_proj_w, b4_proj_bias):
    n, cin, h, w = x_nchw.shape
    c1 = b1_w.shape[1]
    c2r = b2_red_w.shape[1]
    c2 = b2_conv_w.shape[-1]
    c3 = b3_conv_w.shape[-1]
    c4 = b4_proj_w.shape[-1]
    cr = fused1x1_w.shape[1] - c1
    cout = c1 + c2 + c3 + c4
    wn = w * n

    x3 = x_nchw.transpose(2, 3, 0, 1).reshape(h, wn, cin)

    w23, b23, w1e, wcv, w4e, bf = pl.pallas_call(
        _prep_kernel,
        out_shape=(
            jax.ShapeDtypeStruct((cin, cr), jnp.bfloat16),
            jax.ShapeDtypeStruct((1, cr), jnp.float32),
            jax.ShapeDtypeStruct((cin, cout), jnp.bfloat16),
            jax.ShapeDtypeStruct((9, cr, cout), jnp.bfloat16),
            jax.ShapeDtypeStruct((cin, cout), jnp.bfloat16),
            jax.ShapeDtypeStruct((1, cout), jnp.float32),
        ),
        compiler_params=pltpu.CompilerParams(
            vmem_limit_bytes=64 * 1024 * 1024),
    )(fused1x1_w, b2_conv_w.reshape(9, c2r, c2),
      b3_conv_w.reshape(9, cr - c2r, c3), b4_proj_w,
      fused1x1_bias.reshape(1, c1 + cr), b2_conv_bias.reshape(1, c2),
      b3_conv_bias.reshape(1, c3), b4_proj_bias.reshape(1, c4))

    out3 = pl.pallas_call(
        _inception_kernel,
        out_shape=jax.ShapeDtypeStruct((h, wn, cout), jnp.float32),
        grid=(h,),
        in_specs=[
            pl.BlockSpec((1, wn, cin), lambda i: (0, 0, 0)),
            pl.BlockSpec((1, wn, cin),
                         lambda i, _h=h: (jnp.minimum(i + 1, _h - 1), 0, 0)),
            pl.BlockSpec((cin, cr), lambda i: (0, 0)),
            pl.BlockSpec((1, cr), lambda i: (0, 0)),
            pl.BlockSpec((cin, cout), lambda i: (0, 0)),
            pl.BlockSpec((9, cr, cout), lambda i: (0, 0, 0)),
            pl.BlockSpec((cin, cout), lambda i: (0, 0)),
            pl.BlockSpec((1, cout), lambda i: (0, 0)),
        ],
        out_specs=pl.BlockSpec((1, wn, cout), lambda i: (i, 0, 0)),
        scratch_shapes=[pltpu.VMEM((3, wn, cin), jnp.bfloat16),
                        pltpu.VMEM((3, wn, cr), jnp.bfloat16)],
        compiler_params=pltpu.CompilerParams(
            dimension_semantics=("arbitrary",),
            vmem_limit_bytes=64 * 1024 * 1024,
        ),
    )(x3, x3, w23, b23, w1e, wcv, w4e, bf)

    return out3.reshape(h, w, n, cout).transpose(2, 3, 0, 1)


# speedup vs baseline: 2.5466x; 1.0041x over previous
import jax
import jax.numpy as jnp
from jax import lax
from jax.experimental import pallas as pl
from jax.experimental.pallas import tpu as pltpu

_C10 = (((1,), (0,)), ((), ()))


def _dot(a, b):
    return lax.dot_general(a, b, _C10, preferred_element_type=jnp.float32)


def _prep_kernel(fw_ref, w2_ref, w3_ref, w4_ref, fb_ref, b2b_ref, b3b_ref,
                 b4b_ref, w23_ref, b23_ref, w1e_ref, wcv_ref, w4e_ref,
                 bf_ref):
    c1 = fw_ref.shape[1] - w23_ref.shape[1]
    c2r, c2 = w2_ref.shape[1], w2_ref.shape[2]
    c3 = w3_ref.shape[2]
    w23_ref[...] = fw_ref[:, c1:]
    b23_ref[...] = fb_ref[:, c1:]
    w1e_ref[...] = jnp.zeros_like(w1e_ref)
    w1e_ref[:, 0:c1] = fw_ref[:, 0:c1]
    wcv_ref[...] = jnp.zeros_like(wcv_ref)
    for i in range(9):
        wcv_ref[i, 0:c2r, c1:c1 + c2] = w2_ref[i]
        wcv_ref[i, c2r:, c1 + c2:c1 + c2 + c3] = w3_ref[i]
    w4e_ref[...] = jnp.zeros_like(w4e_ref)
    w4e_ref[:, c1 + c2 + c3:] = w4_ref[...]
    bf_ref[0:1, 0:c1] = fb_ref[:, 0:c1]
    bf_ref[0:1, c1:c1 + c2] = b2b_ref[...]
    bf_ref[0:1, c1 + c2:c1 + c2 + c3] = b3b_ref[...]
    bf_ref[0:1, c1 + c2 + c3:] = b4b_ref[...]


def _shift(v, ox, sh, fill):
    if ox == 0:
        return v
    z = jnp.full((sh, v.shape[1]), fill, v.dtype)
    if ox > 0:
        return jnp.concatenate([v[sh:], z], axis=0)
    return jnp.concatenate([z, v[:-sh]], axis=0)


def _inception_kernel(x0_ref, xp_ref, w23_ref, b23_ref, w1e_ref, wcv_ref,
                      w4e_ref, bf_ref, o_ref, xring, yring, hring):
    i = pl.program_id(0)
    nh = pl.num_programs(0)
    sh = o_ref.shape[1] // 28
    ninf = -jnp.inf
    w23 = w23_ref[...]
    b23 = b23_ref[...]

    def red(xb):
        return jnp.maximum(_dot(xb, w23) + b23, 0.0).astype(jnp.bfloat16)

    def hmax(v):
        return jnp.maximum(v, jnp.maximum(_shift(v, 1, sh, ninf),
                                          _shift(v, -1, sh, ninf)))

    @pl.when(i == 0)
    def _():
        xb0 = x0_ref[0].astype(jnp.bfloat16)
        xring[0] = xb0
        yring[0] = red(xb0)
        hring[0] = hmax(xb0)

    s_next = lax.rem(i + 1, 3)
    xb1 = xp_ref[0].astype(jnp.bfloat16)
    xring[s_next] = xb1
    yring[s_next] = red(xb1)
    hring[s_next] = hmax(xb1)

    s_prev = lax.rem(i + 2, 3)
    s_cur = lax.rem(i, 3)
    first = i == 0
    last = i == nh - 1

    yc = yring[s_cur]
    ym = jnp.where(first, jnp.zeros_like(yc), yring[s_prev])
    yp = jnp.where(last, jnp.zeros_like(yc), yring[s_next])
    xbc = xring[s_cur]

    acc = _dot(xbc, w1e_ref[...]) + bf_ref[...]
    t = 0
    for b in (ym, yc, yp):
        for ox in (-1, 0, 1):
            acc = acc + _dot(_shift(b, ox, sh, 0), wcv_ref[t])
            t += 1

    hm = jnp.where(first, jnp.full_like(xbc, ninf), hring[s_prev])
    hp = jnp.where(last, jnp.full_like(xbc, ninf), hring[s_next])
    pooled = jnp.maximum(hring[s_cur], jnp.maximum(hm, hp))
    acc = acc + _dot(pooled, w4e_ref[...])

    o_ref[0] = jnp.maximum(acc, 0.0)


def kernel(x_nchw, b1_w, b2_red_w, b3_red_w, fused1x1_w, fused1x1_bias,
           b2_conv_w, b2_conv_bias, b3_conv_w, b3_conv_bias,
           b4_proj_w, b4_proj_bias):
    n, cin, h, w = x_nchw.shape
    c1 = b1_w.shape[1]
    c2r = b2_red_w.shape[1]
    c2 = b2_conv_w.shape[-1]
    c3 = b3_conv_w.shape[-1]
    c4 = b4_proj_w.shape[-1]
    cr = fused1x1_w.shape[1] - c1
    cout = c1 + c2 + c3 + c4
    wn = w * n

    x3 = x_nchw.transpose(2, 3, 0, 1).reshape(h, wn, cin)

    w23, b23, w1e, wcv, w4e, bf = pl.pallas_call(
        _prep_kernel,
        out_shape=(
            jax.ShapeDtypeStruct((cin, cr), jnp.bfloat16),
            jax.ShapeDtypeStruct((1, cr), jnp.float32),
            jax.ShapeDtypeStruct((cin, cout), jnp.bfloat16),
            jax.ShapeDtypeStruct((9, cr, cout), jnp.bfloat16),
            jax.ShapeDtypeStruct((cin, cout), jnp.bfloat16),
            jax.ShapeDtypeStruct((1, cout), jnp.float32),
        ),
        compiler_params=pltpu.CompilerParams(
            vmem_limit_bytes=64 * 1024 * 1024),
    )(fused1x1_w, b2_conv_w.reshape(9, c2r, c2),
      b3_conv_w.reshape(9, cr - c2r, c3), b4_proj_w,
      fused1x1_bias.reshape(1, c1 + cr), b2_conv_bias.reshape(1, c2),
      b3_conv_bias.reshape(1, c3), b4_proj_bias.reshape(1, c4))

    out3 = pl.pallas_call(
        _inception_kernel,
        out_shape=jax.ShapeDtypeStruct((h, wn, cout), jnp.float32),
        grid=(h,),
        in_specs=[
            pl.BlockSpec((1, wn, cin), lambda i: (0, 0, 0)),
            pl.BlockSpec((1, wn, cin),
                         lambda i, _h=h: (jnp.minimum(i + 1, _h - 1), 0, 0)),
            pl.BlockSpec((cin, cr), lambda i: (0, 0)),
            pl.BlockSpec((1, cr), lambda i: (0, 0)),
            pl.BlockSpec((cin, cout), lambda i: (0, 0)),
            pl.BlockSpec((9, cr, cout), lambda i: (0, 0, 0)),
            pl.BlockSpec((cin, cout), lambda i: (0, 0)),
            pl.BlockSpec((1, cout), lambda i: (0, 0)),
        ],
        out_specs=pl.BlockSpec((1, wn, cout), lambda i: (i, 0, 0)),
        scratch_shapes=[pltpu.VMEM((3, wn, cin), jnp.bfloat16),
                        pltpu.VMEM((3, wn, cr), jnp.bfloat16),
                        pltpu.VMEM((3, wn, cin), jnp.bfloat16)],
        compiler_params=pltpu.CompilerParams(
            dimension_semantics=("arbitrary",),
            vmem_limit_bytes=64 * 1024 * 1024,
        ),
    )(x3, x3, w23, b23, w1e, wcv, w4e, bf)

    return out3.reshape(h, w, n, cout).transpose(2, 3, 0, 1)


# speedup vs baseline: 2.6847x; 1.0542x over previous
import jax
import jax.numpy as jnp
from jax import lax
from jax.experimental import pallas as pl
from jax.experimental.pallas import tpu as pltpu

_C10 = (((1,), (0,)), ((), ()))


def _dot(a, b):
    return lax.dot_general(a, b, _C10, preferred_element_type=jnp.float32)


def _prep_kernel(fw_ref, w2_ref, w3_ref, w4_ref, fb_ref, b2b_ref, b3b_ref,
                 b4b_ref, w23_ref, b23_ref, w1e_ref, wcv_ref, w4e_ref,
                 bf_ref):
    c1 = fw_ref.shape[1] - w23_ref.shape[1]
    c2r, c2 = w2_ref.shape[1], w2_ref.shape[2]
    c3 = w3_ref.shape[2]
    w23_ref[...] = fw_ref[:, c1:]
    b23_ref[...] = fb_ref[:, c1:]
    w1e_ref[...] = jnp.zeros_like(w1e_ref)
    w1e_ref[:, 0:c1] = fw_ref[:, 0:c1]
    cr = w2_ref.shape[1] + w3_ref.shape[1]
    wcv_ref[...] = jnp.zeros_like(wcv_ref)
    for rot in range(3):
        for oxi in range(3):
            for k in range(3):
                dy = (k - rot) % 3
                dy = dy - 3 if dy == 2 else dy
                t = (dy + 1) * 3 + oxi
                base = cr * k
                wcv_ref[rot, oxi, base:base + c2r, c1:c1 + c2] = w2_ref[t]
                wcv_ref[rot, oxi, base + c2r:base + cr,
                        c1 + c2:c1 + c2 + c3] = w3_ref[t]
    w4e_ref[...] = jnp.zeros_like(w4e_ref)
    w4e_ref[:, c1 + c2 + c3:] = w4_ref[...]
    bf_ref[0:1, 0:c1] = fb_ref[:, 0:c1]
    bf_ref[0:1, c1:c1 + c2] = b2b_ref[...]
    bf_ref[0:1, c1 + c2:c1 + c2 + c3] = b3b_ref[...]
    bf_ref[0:1, c1 + c2 + c3:] = b4b_ref[...]


def _shift(v, ox, sh, fill):
    if ox == 0:
        return v
    z = jnp.full((sh, v.shape[1]), fill, v.dtype)
    if ox > 0:
        return jnp.concatenate([v[sh:], z], axis=0)
    return jnp.concatenate([z, v[:-sh]], axis=0)


def _inception_kernel(x0_ref, xp_ref, w23_ref, b23_ref, w1e_ref, wcv_ref,
                      w4e_ref, bf_ref, o_ref, xring, ycat, hring):
    i = pl.program_id(0)
    nh = pl.num_programs(0)
    sh = o_ref.shape[1] // 28
    ninf = -jnp.inf
    w23 = w23_ref[...]
    b23 = b23_ref[...]

    def red(xb):
        return jnp.maximum(_dot(xb, w23) + b23, 0.0).astype(jnp.bfloat16)

    def hmax(v):
        return jnp.maximum(v, jnp.maximum(_shift(v, 1, sh, ninf),
                                          _shift(v, -1, sh, ninf)))

    cr = w23_ref.shape[1]

    @pl.when(i == 0)
    def _():
        xb0 = x0_ref[0].astype(jnp.bfloat16)
        xring[0] = xb0
        ycat[:, 0:cr] = red(xb0)
        ycat[:, 2 * cr:] = jnp.zeros_like(ycat[:, 2 * cr:])
        hring[0] = hmax(xb0)

    s_next = lax.rem(i + 1, 3)
    s_prev = lax.rem(i + 2, 3)
    s_cur = lax.rem(i, 3)
    first = i == 0
    last = i == nh - 1

    xb1 = xp_ref[0].astype(jnp.bfloat16)
    xring[s_next] = xb1
    hring[s_next] = hmax(xb1)
    yv = jnp.where(last, jnp.zeros_like(xb1[:, :cr]), red(xb1))
    for k in range(3):
        @pl.when(s_next == k)
        def _(k=k):
            ycat[:, cr * k:cr * (k + 1)] = yv

    xbc = xring[s_cur]

    acc = _dot(xbc, w1e_ref[...]) + bf_ref[...]
    yall = ycat[...]
    for oxi, ox in enumerate((-1, 0, 1)):
        acc = acc + _dot(_shift(yall, ox, sh, 0), wcv_ref[s_cur, oxi])

    hm = jnp.where(first, jnp.full_like(xbc, ninf), hring[s_prev])
    hp = jnp.where(last, jnp.full_like(xbc, ninf), hring[s_next])
    pooled = jnp.maximum(hring[s_cur], jnp.maximum(hm, hp))
    acc = acc + _dot(pooled, w4e_ref[...])

    o_ref[0] = jnp.maximum(acc, 0.0)


def kernel(x_nchw, b1_w, b2_red_w, b3_red_w, fused1x1_w, fused1x1_bias,
           b2_conv_w, b2_conv_bias, b3_conv_w, b3_conv_bias,
           b4_proj_w, b4_proj_bias):
    n, cin, h, w = x_nchw.shape
    c1 = b1_w.shape[1]
    c2r = b2_red_w.shape[1]
    c2 = b2_conv_w.shape[-1]
    c3 = b3_conv_w.shape[-1]
    c4 = b4_proj_w.shape[-1]
    cr = fused1x1_w.shape[1] - c1
    cout = c1 + c2 + c3 + c4
    wn = w * n

    x3 = x_nchw.transpose(2, 3, 0, 1).reshape(h, wn, cin)

    w23, b23, w1e, wcv, w4e, bf = pl.pallas_call(
        _prep_kernel,
        out_shape=(
            jax.ShapeDtypeStruct((cin, cr), jnp.bfloat16),
            jax.ShapeDtypeStruct((1, cr), jnp.float32),
            jax.ShapeDtypeStruct((cin, cout), jnp.bfloat16),
            jax.ShapeDtypeStruct((3, 3, 3 * cr, cout), jnp.bfloat16),
            jax.ShapeDtypeStruct((cin, cout), jnp.bfloat16),
            jax.ShapeDtypeStruct((1, cout), jnp.float32),
        ),
        compiler_params=pltpu.CompilerParams(
            vmem_limit_bytes=64 * 1024 * 1024),
    )(fused1x1_w, b2_conv_w.reshape(9, c2r, c2),
      b3_conv_w.reshape(9, cr - c2r, c3), b4_proj_w,
      fused1x1_bias.reshape(1, c1 + cr), b2_conv_bias.reshape(1, c2),
      b3_conv_bias.reshape(1, c3), b4_proj_bias.reshape(1, c4))

    out3 = pl.pallas_call(
        _inception_kernel,
        out_shape=jax.ShapeDtypeStruct((h, wn, cout), jnp.float32),
        grid=(h,),
        in_specs=[
            pl.BlockSpec((1, wn, cin), lambda i: (0, 0, 0)),
            pl.BlockSpec((1, wn, cin),
                         lambda i, _h=h: (jnp.minimum(i + 1, _h - 1), 0, 0)),
            pl.BlockSpec((cin, cr), lambda i: (0, 0)),
            pl.BlockSpec((1, cr), lambda i: (0, 0)),
            pl.BlockSpec((cin, cout), lambda i: (0, 0)),
            pl.BlockSpec((3, 3, 3 * cr, cout), lambda i: (0, 0, 0, 0)),
            pl.BlockSpec((cin, cout), lambda i: (0, 0)),
            pl.BlockSpec((1, cout), lambda i: (0, 0)),
        ],
        out_specs=pl.BlockSpec((1, wn, cout), lambda i: (i, 0, 0)),
        scratch_shapes=[pltpu.VMEM((3, wn, cin), jnp.bfloat16),
                        pltpu.VMEM((wn, 3 * cr), jnp.bfloat16),
                        pltpu.VMEM((3, wn, cin), jnp.bfloat16)],
        compiler_params=pltpu.CompilerParams(
            dimension_semantics=("arbitrary",),
            vmem_limit_bytes=64 * 1024 * 1024,
        ),
    )(x3, x3, w23, b23, w1e, wcv, w4e, bf)

    return out3.reshape(h, w, n, cout).transpose(2, 3, 0, 1)


# speedup vs baseline: 2.8033x; 1.0442x over previous
import jax
import jax.numpy as jnp
from jax import lax
from jax.experimental import pallas as pl
from jax.experimental.pallas import tpu as pltpu

_C10 = (((1,), (0,)), ((), ()))


def _dot(a, b):
    return lax.dot_general(a, b, _C10, preferred_element_type=jnp.float32)


def _prep_kernel(fw_ref, w2_ref, w3_ref, w4_ref, fb_ref, b2b_ref, b3b_ref,
                 b4b_ref, w23_ref, b23_ref, w1e_ref, wcv_ref, w4e_ref,
                 bf_ref):
    c1 = fw_ref.shape[1] - w23_ref.shape[1]
    c2r, c2 = w2_ref.shape[1], w2_ref.shape[2]
    c3 = w3_ref.shape[2]
    w23_ref[...] = fw_ref[:, c1:]
    b23_ref[...] = fb_ref[:, c1:]
    w1e_ref[...] = jnp.zeros_like(w1e_ref)
    w1e_ref[:, 0:c1] = fw_ref[:, 0:c1]
    cr = w2_ref.shape[1] + w3_ref.shape[1]
    cs = wcv_ref.shape[2] // 3
    wcv_ref[...] = jnp.zeros_like(wcv_ref)
    for rot in range(3):
        for oxi in range(3):
            for k in range(3):
                dy = (k - rot) % 3
                dy = dy - 3 if dy == 2 else dy
                t = (dy + 1) * 3 + oxi
                base = cs * k
                wcv_ref[rot, oxi, base:base + c2r, c1:c1 + c2] = w2_ref[t]
                wcv_ref[rot, oxi, base + c2r:base + cr,
                        c1 + c2:c1 + c2 + c3] = w3_ref[t]
    w4e_ref[...] = jnp.zeros_like(w4e_ref)
    w4e_ref[:, c1 + c2 + c3:] = w4_ref[...]
    bf_ref[0:1, 0:c1] = fb_ref[:, 0:c1]
    bf_ref[0:1, c1:c1 + c2] = b2b_ref[...]
    bf_ref[0:1, c1 + c2:c1 + c2 + c3] = b3b_ref[...]
    bf_ref[0:1, c1 + c2 + c3:] = b4b_ref[...]


def _shift(v, ox, sh, fill):
    if ox == 0:
        return v
    z = jnp.full((sh, v.shape[1]), fill, v.dtype)
    if ox > 0:
        return jnp.concatenate([v[sh:], z], axis=0)
    return jnp.concatenate([z, v[:-sh]], axis=0)


def _inception_kernel(x0_ref, xp_ref, w23_ref, b23_ref, w1e_ref, wcv_ref,
                      w4e_ref, bf_ref, o_ref, xring, ycat, hring):
    i = pl.program_id(0)
    nh = pl.num_programs(0)
    sh = o_ref.shape[1] // 28
    ninf = -jnp.inf
    w23 = w23_ref[...]
    b23 = b23_ref[...]

    def red(xb):
        return jnp.maximum(_dot(xb, w23) + b23, 0.0).astype(jnp.bfloat16)

    def hmax(v):
        return jnp.maximum(v, jnp.maximum(_shift(v, 1, sh, ninf),
                                          _shift(v, -1, sh, ninf)))

    cr = w23_ref.shape[1]
    cs = ycat.shape[1] // 3

    @pl.when(i == 0)
    def _():
        xb0 = x0_ref[0].astype(jnp.bfloat16)
        xring[0] = xb0
        ycat[...] = jnp.zeros_like(ycat)
        ycat[:, 0:cr] = red(xb0)
        hring[0] = hmax(xb0)

    s_next = lax.rem(i + 1, 3)
    s_prev = lax.rem(i + 2, 3)
    s_cur = lax.rem(i, 3)
    first = i == 0
    last = i == nh - 1

    xb1 = xp_ref[0].astype(jnp.bfloat16)
    xring[s_next] = xb1
    hring[s_next] = hmax(xb1)
    yv = jnp.where(last, jnp.zeros_like(xb1[:, :cr]), red(xb1))
    for k in range(3):
        @pl.when(s_next == k)
        def _(k=k):
            ycat[:, cs * k:cs * k + cr] = yv

    xbc = xring[s_cur]

    acc = _dot(xbc, w1e_ref[...]) + bf_ref[...]
    yall = ycat[...]
    for oxi, ox in enumerate((-1, 0, 1)):
        acc = acc + _dot(_shift(yall, ox, sh, 0), wcv_ref[s_cur, oxi])

    hm = jnp.where(first, jnp.full_like(xbc, ninf), hring[s_prev])
    hp = jnp.where(last, jnp.full_like(xbc, ninf), hring[s_next])
    pooled = jnp.maximum(hring[s_cur], jnp.maximum(hm, hp))
    acc = acc + _dot(pooled, w4e_ref[...])

    o_ref[0] = jnp.maximum(acc, 0.0)


def kernel(x_nchw, b1_w, b2_red_w, b3_red_w, fused1x1_w, fused1x1_bias,
           b2_conv_w, b2_conv_bias, b3_conv_w, b3_conv_bias,
           b4_proj_w, b4_proj_bias):
    n, cin, h, w = x_nchw.shape
    c1 = b1_w.shape[1]
    c2r = b2_red_w.shape[1]
    c2 = b2_conv_w.shape[-1]
    c3 = b3_conv_w.shape[-1]
    c4 = b4_proj_w.shape[-1]
    cr = fused1x1_w.shape[1] - c1
    cout = c1 + c2 + c3 + c4
    cs = ((cr + 127) // 128) * 128
    wn = w * n

    x3 = x_nchw.transpose(2, 3, 0, 1).reshape(h, wn, cin)

    w23, b23, w1e, wcv, w4e, bf = pl.pallas_call(
        _prep_kernel,
        out_shape=(
            jax.ShapeDtypeStruct((cin, cr), jnp.bfloat16),
            jax.ShapeDtypeStruct((1, cr), jnp.float32),
            jax.ShapeDtypeStruct((cin, cout), jnp.bfloat16),
            jax.ShapeDtypeStruct((3, 3, 3 * cs, cout), jnp.bfloat16),
            jax.ShapeDtypeStruct((cin, cout), jnp.bfloat16),
            jax.ShapeDtypeStruct((1, cout), jnp.float32),
        ),
        compiler_params=pltpu.CompilerParams(
            vmem_limit_bytes=64 * 1024 * 1024),
    )(fused1x1_w, b2_conv_w.reshape(9, c2r, c2),
      b3_conv_w.reshape(9, cr - c2r, c3), b4_proj_w,
      fused1x1_bias.reshape(1, c1 + cr), b2_conv_bias.reshape(1, c2),
      b3_conv_bias.reshape(1, c3), b4_proj_bias.reshape(1, c4))

    out3 = pl.pallas_call(
        _inception_kernel,
        out_shape=jax.ShapeDtypeStruct((h, wn, cout), jnp.float32),
        grid=(h,),
        in_specs=[
            pl.BlockSpec((1, wn, cin), lambda i: (0, 0, 0)),
            pl.BlockSpec((1, wn, cin),
                         lambda i, _h=h: (jnp.minimum(i + 1, _h - 1), 0, 0)),
            pl.BlockSpec((cin, cr), lambda i: (0, 0)),
            pl.BlockSpec((1, cr), lambda i: (0, 0)),
            pl.BlockSpec((cin, cout), lambda i: (0, 0)),
            pl.BlockSpec((3, 3, 3 * cs, cout), lambda i: (0, 0, 0, 0)),
            pl.BlockSpec((cin, cout), lambda i: (0, 0)),
            pl.BlockSpec((1, cout), lambda i: (0, 0)),
        ],
        out_specs=pl.BlockSpec((1, wn, cout), lambda i: (i, 0, 0)),
        scratch_shapes=[pltpu.VMEM((3, wn, cin), jnp.bfloat16),
                        pltpu.VMEM((wn, 3 * cs), jnp.bfloat16),
                        pltpu.VMEM((3, wn, cin), jnp.bfloat16)],
        compiler_params=pltpu.CompilerParams(
            dimension_semantics=("arbitrary",),
            vmem_limit_bytes=64 * 1024 * 1024,
        ),
    )(x3, x3, w23, b23, w1e, wcv, w4e, bf)

    return out3.reshape(h, w, n, cout).transpose(2, 3, 0, 1)
